# baseline (speedup 1.0000x reference)
import functools

import jax
import jax.numpy as jnp
from jax import lax
from jax.experimental import pallas as pl
from jax.experimental.pallas import tpu as pltpu
from jax.experimental.pallas import tpu_sc as plsc

_DIM = 256
_K = 1024
_TOK_BLK = 256
_K_CHUNK = 128


def _vq_body(x_ref, cb_ref, cb2_ref, idx_ref, comm_ref, perp_ref,
             counts_scr, csum_scr):
    b = pl.program_id(0)
    tc = pl.program_id(1)
    nb = pl.num_programs(0)
    ntc = pl.num_programs(1)
    first = jnp.logical_and(b == 0, tc == 0)
    last = jnp.logical_and(b == nb - 1, tc == ntc - 1)

    xT = x_ref[0]

    mm2T = jnp.dot(cb2_ref[...], xT, preferred_element_type=jnp.float32)
    xsq = jnp.sum(xT * xT, axis=0)

    m = None
    idx = None
    for kc in range(_K // _K_CHUNK):
        sl = pl.ds(kc * _K_CHUNK, _K_CHUNK)
        cb_c = cb_ref[sl, :]
        cbn_c = jnp.sum(cb_c * cb_c, axis=1)
        d_c = (xsq[None, :] - mm2T[kc * _K_CHUNK:(kc + 1) * _K_CHUNK, :]) \
            + cbn_c[:, None]
        cmin = jnp.min(d_c, axis=0)
        iota_c = jax.lax.broadcasted_iota(jnp.int32, (_K_CHUNK, _TOK_BLK), 0) \
            + kc * _K_CHUNK
        idxc = jnp.min(jnp.where(d_c == cmin[None, :], iota_c, _K), axis=0)
        if kc == 0:
            m, idx = cmin, idxc
        else:
            idx = jnp.where(cmin < m, idxc, idx)
            m = jnp.minimum(m, cmin)
    idx_ref[0, 0, :] = idx

    iota = jax.lax.broadcasted_iota(jnp.int32, (_K, _TOK_BLK), 0)
    onehot = (iota == idx[None, :]).astype(jnp.float32)
    blk_csum = jnp.sum(m)

    @pl.when(first)
    def _():
        counts_scr[...] = onehot
        csum_scr[0] = blk_csum

    @pl.when(jnp.logical_not(first))
    def _():
        counts_scr[...] = counts_scr[...] + onehot
        csum_scr[0] = csum_scr[0] + blk_csum

    @pl.when(last)
    def _():
        counts = jnp.sum(counts_scr[...], axis=1)
        total = jnp.sum(counts)
        probs = counts / jnp.maximum(total, 1.0)
        ent = -jnp.sum(probs * jnp.log(probs + 1e-10))
        perp_ref[...] = jnp.full((1, 1), jnp.exp(ent), jnp.float32)
        n_elems = nb * ntc * _TOK_BLK * _DIM
        comm_ref[...] = jnp.full((1, 1), csum_scr[0] / n_elems, jnp.float32)

    @pl.when(jnp.logical_not(last))
    def _():
        perp_ref[...] = jnp.zeros((1, 1), jnp.float32)
        comm_ref[...] = jnp.zeros((1, 1), jnp.float32)


def _tc_argmin(x, codebook):
    b, d, t = x.shape
    n_tc = t // _TOK_BLK
    grid = (b, n_tc)

    idx3, comm, perp = pl.pallas_call(
        _vq_body,
        grid=grid,
        in_specs=[
            pl.BlockSpec((1, d, _TOK_BLK), lambda i, j: (i, 0, j)),
            pl.BlockSpec((_K, d), lambda i, j: (0, 0)),
            pl.BlockSpec((_K, d), lambda i, j: (0, 0)),
        ],
        out_specs=[
            pl.BlockSpec((1, 1, _TOK_BLK), lambda i, j: (i, 0, j)),
            pl.BlockSpec((1, 1), lambda i, j: (0, 0)),
            pl.BlockSpec((1, 1), lambda i, j: (0, 0)),
        ],
        out_shape=[
            jax.ShapeDtypeStruct((b, 1, t), jnp.int32),
            jax.ShapeDtypeStruct((1, 1), jnp.float32),
            jax.ShapeDtypeStruct((1, 1), jnp.float32),
        ],
        scratch_shapes=[
            pltpu.VMEM((_K, _TOK_BLK), jnp.float32),
            pltpu.SMEM((1,), jnp.float32),
        ],
        compiler_params=pltpu.CompilerParams(
            dimension_semantics=("arbitrary", "arbitrary"),
        ),
    )(x, codebook, codebook * 2.0)
    return idx3, comm, perp


_SC_CHUNK = 128


def _make_sc_gather(n_rows, dim):
    info = plsc.get_sparse_core_info()
    nw = info.num_cores * info.num_subcores
    rows_per_w = n_rows // nw
    n_chunks = rows_per_w // _SC_CHUNK
    mesh = plsc.VectorSubcoreMesh(core_axis_name="c", subcore_axis_name="s")

    @functools.partial(
        pl.kernel,
        mesh=mesh,
        out_type=jax.ShapeDtypeStruct((n_rows, dim), jnp.float32),
        scratch_types=[
            pltpu.VMEM((_SC_CHUNK,), jnp.int32),
            pltpu.VMEM((_SC_CHUNK, dim), jnp.float32),
            pltpu.SemaphoreType.DMA,
        ],
    )
    def sc_gather(table_hbm, idx_hbm, out_hbm, idx_v, rows_v, sem):
        wid = lax.axis_index("s") * info.num_cores + lax.axis_index("c")
        base = wid * rows_per_w
        for c in range(n_chunks):
            off = base + c * _SC_CHUNK
            pltpu.sync_copy(idx_hbm.at[pl.ds(off, _SC_CHUNK)], idx_v)
            pltpu.async_copy(table_hbm.at[idx_v], rows_v, sem).wait()
            pltpu.sync_copy(rows_v, out_hbm.at[pl.ds(off, _SC_CHUNK)])

    return sc_gather


def kernel(x, codebook):
    b, d, t = x.shape
    idx3, comm, perp = _tc_argmin(x, codebook)
    idx_flat = idx3.reshape(-1)

    q_rows = _make_sc_gather(b * t, d)(codebook, idx_flat)
    quantized = jnp.transpose(q_rows.reshape(b, t, d), (0, 2, 1))

    indices_2d = idx3.reshape(b, t)
    codebook_loss = jnp.zeros((), dtype=jnp.float32)
    return (quantized, indices_2d, codebook_loss, comm.reshape(()), perp.reshape(()))


# speedup vs baseline: 1.4727x; 1.4727x over previous
import jax
import jax.numpy as jnp
from jax.experimental import pallas as pl
from jax.experimental.pallas import tpu as pltpu

_DIM = 256
_K = 1024
_TOK_BLK = 256
_K_CHUNK = 128


def _vq_body(x_ref, cb_ref, cb2_ref, cbb_ref, q_ref, idx_ref, comm_ref,
             perp_ref, counts_scr, csum_scr):
    b = pl.program_id(0)
    tc = pl.program_id(1)
    nb = pl.num_programs(0)
    ntc = pl.num_programs(1)
    first = jnp.logical_and(b == 0, tc == 0)
    last = jnp.logical_and(b == nb - 1, tc == ntc - 1)

    xT = x_ref[0]
    cb = cb_ref[...]

    mm2T = jnp.dot(cb2_ref[...], xT, preferred_element_type=jnp.float32)
    xsq = jnp.sum(xT * xT, axis=0)

    m = None
    idx = None
    for kc in range(_K // _K_CHUNK):
        sl = pl.ds(kc * _K_CHUNK, _K_CHUNK)
        cb_c = cb_ref[sl, :]
        cbn_c = jnp.sum(cb_c * cb_c, axis=1)
        d_c = (xsq[None, :] - mm2T[kc * _K_CHUNK:(kc + 1) * _K_CHUNK, :]) \
            + cbn_c[:, None]
        cmin = jnp.min(d_c, axis=0)
        iota_c = jax.lax.broadcasted_iota(jnp.int32, (_K_CHUNK, _TOK_BLK), 0) \
            + kc * _K_CHUNK
        idxc = jnp.min(jnp.where(d_c == cmin[None, :], iota_c, _K), axis=0)
        if kc == 0:
            m, idx = cmin, idxc
        else:
            idx = jnp.where(cmin < m, idxc, idx)
            m = jnp.minimum(m, cmin)
    idx_ref[0, 0, :] = idx

    iota = jax.lax.broadcasted_iota(jnp.int32, (_K, _TOK_BLK), 0)

    hit = iota == idx[None, :]
    onehot = hit.astype(jnp.float32)
    qT = jax.lax.dot_general(cbb_ref[...], hit.astype(jnp.bfloat16),
                             (((0,), (0,)), ((), ())),
                             preferred_element_type=jnp.float32)
    q_ref[0] = qT

    blk_csum = jnp.sum(m)

    @pl.when(first)
    def _():
        counts_scr[...] = onehot
        csum_scr[0] = blk_csum

    @pl.when(jnp.logical_not(first))
    def _():
        counts_scr[...] = counts_scr[...] + onehot
        csum_scr[0] = csum_scr[0] + blk_csum

    @pl.when(last)
    def _():
        counts = jnp.sum(counts_scr[...], axis=1)
        total = jnp.sum(counts)
        probs = counts / jnp.maximum(total, 1.0)
        ent = -jnp.sum(probs * jnp.log(probs + 1e-10))
        perp_ref[...] = jnp.full((1, 1), jnp.exp(ent), jnp.float32)
        n_elems = nb * ntc * _TOK_BLK * _DIM
        comm_ref[...] = jnp.full((1, 1), csum_scr[0] / n_elems, jnp.float32)

    @pl.when(jnp.logical_not(last))
    def _():
        perp_ref[...] = jnp.zeros((1, 1), jnp.float32)
        comm_ref[...] = jnp.zeros((1, 1), jnp.float32)


def kernel(x, codebook):
    b, d, t = x.shape
    n_tc = t // _TOK_BLK
    grid = (b, n_tc)

    q, idx3, comm, perp = pl.pallas_call(
        _vq_body,
        grid=grid,
        in_specs=[
            pl.BlockSpec((1, d, _TOK_BLK), lambda i, j: (i, 0, j)),
            pl.BlockSpec((_K, d), lambda i, j: (0, 0)),
            pl.BlockSpec((_K, d), lambda i, j: (0, 0)),
            pl.BlockSpec((_K, d), lambda i, j: (0, 0)),
        ],
        out_specs=[
            pl.BlockSpec((1, d, _TOK_BLK), lambda i, j: (i, 0, j)),
            pl.BlockSpec((1, 1, _TOK_BLK), lambda i, j: (i, 0, j)),
            pl.BlockSpec((1, 1), lambda i, j: (0, 0)),
            pl.BlockSpec((1, 1), lambda i, j: (0, 0)),
        ],
        out_shape=[
            jax.ShapeDtypeStruct((b, d, t), jnp.float32),
            jax.ShapeDtypeStruct((b, 1, t), jnp.int32),
            jax.ShapeDtypeStruct((1, 1), jnp.float32),
            jax.ShapeDtypeStruct((1, 1), jnp.float32),
        ],
        scratch_shapes=[
            pltpu.VMEM((_K, _TOK_BLK), jnp.float32),
            pltpu.SMEM((1,), jnp.float32),
        ],
        compiler_params=pltpu.CompilerParams(
            dimension_semantics=("arbitrary", "arbitrary"),
        ),
    )(x, codebook, codebook * 2.0, codebook.astype(jnp.bfloat16))

    indices_2d = idx3.reshape(b, t)
    codebook_loss = jnp.zeros((), dtype=jnp.float32)
    return (q, indices_2d, codebook_loss, comm.reshape(()), perp.reshape(()))
